# baseline (speedup 1.0000x reference)
import functools

import numpy as np
import jax
import jax.numpy as jnp
from jax import lax
from jax.experimental import pallas as pl
from jax.experimental.pallas import tpu as pltpu
from jax.experimental.pallas import tpu_sc as plsc

B = 4096
NF = 26
V = 100000
D = 64
NI = NF + 1
TB = 256

_LI = np.array([i for i in range(NI) for j in range(i)])
_LJ = np.array([j for i in range(NI) for j in range(i)])


def _sc_gather(tables_p, q_idx, nf):
    info = plsc.get_sparse_core_info()
    nc, ns = info.num_cores, info.num_subcores
    nw = nc * ns
    cb = B // nw
    mesh = plsc.VectorSubcoreMesh(core_axis_name="c", subcore_axis_name="s")

    @functools.partial(
        pl.kernel,
        mesh=mesh,
        out_type=jax.ShapeDtypeStruct((nf, B, 2 * D), jnp.float32),
        scratch_types=[
            pltpu.VMEM((cb,), jnp.int32),
            pltpu.VMEM((cb, 2 * D), jnp.float32),
            pltpu.SemaphoreType.DMA,
        ],
        compiler_params=pltpu.CompilerParams(use_tc_tiling_on_sc=False),
    )
    def k(tab_hbm, idx_hbm, out_hbm, idx_v, rows_v, sem):
        wid = lax.axis_index("s") * nc + lax.axis_index("c")
        base = wid * cb

        def body(f, carry):
            pltpu.sync_copy(idx_hbm.at[f, pl.ds(base, cb)], idx_v)
            pltpu.async_copy(tab_hbm.at[f].at[idx_v], rows_v, sem).wait()
            pltpu.sync_copy(rows_v, out_hbm.at[f, pl.ds(base, cb)])
            return carry

        lax.fori_loop(0, nf, body, 0)

    return k(tables_p, q_idx)


def _tc_body(dx_ref, lyp0, lyp1, lyp2, lyp3, lyp4, lyp5, par_ref,
             WbT0, WbT1, WbT2, bb0, bb1, bb2,
             Wt0xT, WpairT, bt0, Wt1T, bt1, Wt2T, bt2, out_ref):
    f32 = jnp.float32
    dot = functools.partial(jnp.dot, preferred_element_type=f32)
    x = jnp.maximum(dot(dx_ref[...], WbT0[...]) + bb0[...], 0.0)
    x = jnp.maximum(dot(x, WbT1[...]) + bb1[...], 0.0)
    x = jnp.maximum(dot(x, WbT2[...]) + bb2[...], 0.0)
    lyp = jnp.concatenate(
        [r[...] for r in (lyp0, lyp1, lyp2, lyp3, lyp4, lyp5)],
        axis=0)
    par = par_ref[...]
    ly = jnp.where(par == 1, lyp[:, :, D:], lyp[:, :, :D])
    T2 = jnp.concatenate([x[None], ly], axis=0)
    Z = lax.dot_general(T2, T2, (((2,), (2,)), ((1,), (1,))),
                        preferred_element_type=f32)
    Zf = Z.reshape(TB, NI * NI)
    z = dot(x, Wt0xT[...]) + dot(Zf, WpairT[...]) + bt0[...]
    z = jnp.maximum(z, 0.0)
    z = jnp.maximum(dot(z, Wt1T[...]) + bt1[...], 0.0)
    z = dot(z, Wt2T[...]) + bt2[...]
    out_ref[...] = 1.0 / (1.0 + jnp.exp(-z))


def _tc_fused(dx, lyps, par, *weights):
    n_tiles = B // TB

    def full(a):
        return pl.BlockSpec(a.shape, lambda i: (0,) * a.ndim)

    in_specs = [
        pl.BlockSpec((TB, 13), lambda i: (i, 0)),
    ] + [
        pl.BlockSpec((c.shape[0], TB, 2 * D), lambda i: (0, i, 0))
        for c in lyps
    ] + [
        pl.BlockSpec((NF, TB, 1), lambda i: (0, i, 0)),
    ] + [full(w) for w in weights]

    return pl.pallas_call(
        _tc_body,
        grid=(n_tiles,),
        in_specs=in_specs,
        out_specs=pl.BlockSpec((TB, 1), lambda i: (i, 0)),
        out_shape=jax.ShapeDtypeStruct((B, 1), jnp.float32),
        compiler_params=pltpu.CompilerParams(
            dimension_semantics=("arbitrary",),
        ),
    )(dx, *lyps, par, *weights)


def kernel(dense_x, lS_o, lS_i, tables, Wb0, bb0, Wb1, bb1, Wb2, bb2,
           Wt0, bt0, Wt1, bt1, Wt2, bt2):
    del lS_o
    tables_p = tables.reshape(NF, V // 2, 2 * D)
    q_idx = lax.shift_right_logical(lS_i, 1)
    par = lax.bitwise_and(lS_i, 1)[:, :, None]
    bounds = [0, 5, 10, 14, 18, 22, 26]
    lyps = [
        _sc_gather(tables_p[f0:f1], q_idx[f0:f1], f1 - f0)
        for f0, f1 in zip(bounds[:-1], bounds[1:])
    ]

    WpairT = jnp.zeros((NI * NI, 512), jnp.float32).at[_LI * NI + _LJ, :].set(
        Wt0[:, D:].T)
    return _tc_fused(
        dense_x, lyps, par,
        Wb0.T, Wb1.T, Wb2.T,
        bb0[None, :], bb1[None, :], bb2[None, :],
        Wt0[:, :D].T, WpairT, bt0[None, :],
        Wt1.T, bt1[None, :], Wt2.T, bt2[None, :],
    )


# speedup vs baseline: 1.2046x; 1.2046x over previous
import functools

import numpy as np
import jax
import jax.numpy as jnp
from jax import lax
from jax.experimental import pallas as pl
from jax.experimental.pallas import tpu as pltpu
from jax.experimental.pallas import tpu_sc as plsc

B = 4096
NF = 26
V = 100000
D = 64
NI = NF + 1
TB = 256

_LI = np.array([i for i in range(NI) for j in range(i)])
_LJ = np.array([j for i in range(NI) for j in range(i)])


def _sc_gather(tables_p, q_idx, nf):
    info = plsc.get_sparse_core_info()
    nc, ns = info.num_cores, info.num_subcores
    nw = nc * ns
    cb = B // nw
    mesh = plsc.VectorSubcoreMesh(core_axis_name="c", subcore_axis_name="s")

    @functools.partial(
        pl.kernel,
        mesh=mesh,
        out_type=jax.ShapeDtypeStruct((nf, B, 2 * D), jnp.float32),
        scratch_types=[
            pltpu.VMEM((cb,), jnp.int32),
            pltpu.VMEM((cb, 2 * D), jnp.float32),
            pltpu.SemaphoreType.DMA,
        ],
        compiler_params=pltpu.CompilerParams(use_tc_tiling_on_sc=False),
    )
    def k(tab_hbm, idx_hbm, out_hbm, idx_v, rows_v, sem):
        wid = lax.axis_index("s") * nc + lax.axis_index("c")
        base = wid * cb

        def body(f, carry):
            pltpu.sync_copy(idx_hbm.at[f, pl.ds(base, cb)], idx_v)
            pltpu.async_copy(tab_hbm.at[f].at[idx_v], rows_v, sem).wait()
            pltpu.sync_copy(rows_v, out_hbm.at[f, pl.ds(base, cb)])
            return carry

        lax.fori_loop(0, nf, body, 0)

    return k(tables_p, q_idx)


VC = 2000


def _depad_body(lo_ref, hi_ref, out_ref):
    out_ref[0] = jnp.concatenate([lo_ref[0], hi_ref[0]], axis=1)


def _tc_depad(tables):
    return pl.pallas_call(
        _depad_body,
        grid=(NF, V // 2 // VC),
        in_specs=[
            pl.BlockSpec((1, VC, D), lambda f, c: (f, c, 0)),
            pl.BlockSpec((1, VC, D), lambda f, c: (f, c + V // 2 // VC, 0)),
        ],
        out_specs=pl.BlockSpec((1, VC, 2 * D), lambda f, c: (f, c, 0)),
        out_shape=jax.ShapeDtypeStruct((NF, V // 2, 2 * D), jnp.float32),
        compiler_params=pltpu.CompilerParams(
            dimension_semantics=("arbitrary", "arbitrary"),
        ),
    )(tables, tables)


def _tc_body(dx_ref, lyp_ref, par_ref, WbT0, WbT1, WbT2, bb0, bb1, bb2,
             Wt0xT, WpairT, bt0, Wt1T, bt1, Wt2T, bt2, out_ref):
    f32 = jnp.float32
    dot = functools.partial(jnp.dot, preferred_element_type=f32)
    x = jnp.maximum(dot(dx_ref[...], WbT0[...]) + bb0[...], 0.0)
    x = jnp.maximum(dot(x, WbT1[...]) + bb1[...], 0.0)
    x = jnp.maximum(dot(x, WbT2[...]) + bb2[...], 0.0)
    lyp = lyp_ref[...]
    par = par_ref[...]
    ly = jnp.where(par == 1, lyp[:, :, D:], lyp[:, :, :D])
    T2 = jnp.concatenate([x[None], ly], axis=0)
    Z = lax.dot_general(T2, T2, (((2,), (2,)), ((1,), (1,))),
                        preferred_element_type=f32)
    Zf = Z.reshape(TB, NI * NI)
    z = dot(x, Wt0xT[...]) + dot(Zf, WpairT[...]) + bt0[...]
    z = jnp.maximum(z, 0.0)
    z = jnp.maximum(dot(z, Wt1T[...]) + bt1[...], 0.0)
    z = dot(z, Wt2T[...]) + bt2[...]
    out_ref[...] = 1.0 / (1.0 + jnp.exp(-z))


def _tc_fused(dx, lyp, par, *weights):
    n_tiles = B // TB

    def full(a):
        return pl.BlockSpec(a.shape, lambda i: (0,) * a.ndim)

    in_specs = [
        pl.BlockSpec((TB, 13), lambda i: (i, 0)),
        pl.BlockSpec((NF, TB, 2 * D), lambda i: (0, i, 0)),
        pl.BlockSpec((NF, TB, 1), lambda i: (0, i, 0)),
    ] + [full(w) for w in weights]

    return pl.pallas_call(
        _tc_body,
        grid=(n_tiles,),
        in_specs=in_specs,
        out_specs=pl.BlockSpec((TB, 1), lambda i: (i, 0)),
        out_shape=jax.ShapeDtypeStruct((B, 1), jnp.float32),
        compiler_params=pltpu.CompilerParams(
            dimension_semantics=("arbitrary",),
        ),
    )(dx, lyp, par, *weights)


def kernel(dense_x, lS_o, lS_i, tables, Wb0, bb0, Wb1, bb1, Wb2, bb2,
           Wt0, bt0, Wt1, bt1, Wt2, bt2):
    del lS_o
    half = V // 2
    q_idx = jnp.where(lS_i < half, lS_i, lS_i - half)
    par = (lS_i >= half).astype(jnp.int32)[:, :, None]
    lyp = _sc_gather(_tc_depad(tables), q_idx, NF)

    WpairT = jnp.zeros((NI * NI, 512), jnp.float32).at[_LI * NI + _LJ, :].set(
        Wt0[:, D:].T)
    return _tc_fused(
        dense_x, lyp, par,
        Wb0.T, Wb1.T, Wb2.T,
        bb0[None, :], bb1[None, :], bb2[None, :],
        Wt0[:, :D].T, WpairT, bt0[None, :],
        Wt1.T, bt1[None, :], Wt2.T, bt2[None, :],
    )


# speedup vs baseline: 1.3741x; 1.1407x over previous
import functools

import numpy as np
import jax
import jax.numpy as jnp
from jax import lax
from jax.experimental import pallas as pl
from jax.experimental.pallas import tpu as pltpu
from jax.experimental.pallas import tpu_sc as plsc

B = 4096
NF = 26
V = 100000
D = 64
NI = NF + 1
TB = 256

_LI = np.array([i for i in range(NI) for j in range(i)])
_LJ = np.array([j for i in range(NI) for j in range(i)])


def _sc_gather(tables, lS_i):
    info = plsc.get_sparse_core_info()
    nc, ns = info.num_cores, info.num_subcores
    nw = nc * ns
    cb = B // nw
    mesh = plsc.VectorSubcoreMesh(core_axis_name="c", subcore_axis_name="s")

    @functools.partial(
        pl.kernel,
        mesh=mesh,
        out_type=jax.ShapeDtypeStruct((NF, B, D), jnp.float32),
        scratch_types=[
            pltpu.VMEM((cb,), jnp.int32),
            pltpu.VMEM((cb, D), jnp.float32),
            pltpu.SemaphoreType.DMA,
        ],
        compiler_params=pltpu.CompilerParams(use_tc_tiling_on_sc=False),
    )
    def k(tab_hbm, idx_hbm, out_hbm, idx_v, rows_v, sem):
        wid = lax.axis_index("s") * nc + lax.axis_index("c")
        base = wid * cb

        def body(f, carry):
            pltpu.sync_copy(idx_hbm.at[f, pl.ds(base, cb)], idx_v)
            pltpu.async_copy(tab_hbm.at[f].at[idx_v], rows_v, sem).wait()
            pltpu.sync_copy(rows_v, out_hbm.at[f, pl.ds(base, cb)])
            return carry

        lax.fori_loop(0, NF, body, 0)

    return k(tables, lS_i)


def _tc_body(dx_ref, ly_ref, WbT0, WbT1, WbT2, bb0, bb1, bb2,
             Wt0xT, WpairT, bt0, Wt1T, bt1, Wt2T, bt2, out_ref):
    f32 = jnp.float32
    dot = functools.partial(jnp.dot, preferred_element_type=f32)
    x = jnp.maximum(dot(dx_ref[...], WbT0[...]) + bb0[...], 0.0)
    x = jnp.maximum(dot(x, WbT1[...]) + bb1[...], 0.0)
    x = jnp.maximum(dot(x, WbT2[...]) + bb2[...], 0.0)
    T2 = jnp.concatenate([x[None], ly_ref[...]], axis=0)
    Z = lax.dot_general(T2, T2, (((2,), (2,)), ((1,), (1,))),
                        preferred_element_type=f32)
    Zf = Z.reshape(TB, NI * NI)
    z = dot(x, Wt0xT[...]) + dot(Zf, WpairT[...]) + bt0[...]
    z = jnp.maximum(z, 0.0)
    z = jnp.maximum(dot(z, Wt1T[...]) + bt1[...], 0.0)
    z = dot(z, Wt2T[...]) + bt2[...]
    out_ref[...] = 1.0 / (1.0 + jnp.exp(-z))


def _tc_fused(dx, ly, *weights):
    n_tiles = B // TB

    def full(a):
        return pl.BlockSpec(a.shape, lambda i: (0,) * a.ndim)

    in_specs = [
        pl.BlockSpec((TB, 13), lambda i: (i, 0)),
        pl.BlockSpec((NF, TB, D), lambda i: (0, i, 0)),
    ] + [full(w) for w in weights]

    return pl.pallas_call(
        _tc_body,
        grid=(n_tiles,),
        in_specs=in_specs,
        out_specs=pl.BlockSpec((TB, 1), lambda i: (i, 0)),
        out_shape=jax.ShapeDtypeStruct((B, 1), jnp.float32),
        compiler_params=pltpu.CompilerParams(
            dimension_semantics=("arbitrary",),
        ),
    )(dx, ly, *weights)


def kernel(dense_x, lS_o, lS_i, tables, Wb0, bb0, Wb1, bb1, Wb2, bb2,
           Wt0, bt0, Wt1, bt1, Wt2, bt2):
    del lS_o
    ly = _sc_gather(tables, lS_i)

    WpairT = jnp.zeros((NI * NI, 512), jnp.float32).at[_LI * NI + _LJ, :].set(
        Wt0[:, D:].T)
    return _tc_fused(
        dense_x, ly,
        Wb0.T, Wb1.T, Wb2.T,
        bb0[None, :], bb1[None, :], bb2[None, :],
        Wt0[:, :D].T, WpairT, bt0[None, :],
        Wt1.T, bt1[None, :], Wt2.T, bt2[None, :],
    )
